# baseline (speedup 1.0000x reference)
import jax
import jax.numpy as jnp
from jax.experimental import pallas as pl
from jax.experimental.pallas import tpu as pltpu
from jax.experimental.pallas import tpu_sc as plsc

_N = 20000
_NPAD = 20480
_ROWS = _NPAD // 128
_K = 6000
_OUT = 1000
_IOU_T = 0.7
_DEAD = -3.0e38

_CN = 6144
_CROWS = _CN // 128
_CPAD = 6400
_TRASHP = 6336
_RFLAT = _CPAD * 4
_TRASHR = 25592
_NWK = 16
_WR = _ROWS // _NWK
_FILL = _CPAD // _NWK


def _select_body(s_ref, sw_ref):
    s = s_ref[...]
    sm = jnp.where(s > 0.0, s, -jnp.inf)
    sbits = jax.lax.bitcast_convert_type(sm, jnp.int32)

    ir = jax.lax.broadcasted_iota(jnp.int32, (_ROWS, 128), 0)
    ic = jax.lax.broadcasted_iota(jnp.int32, (_ROWS, 128), 1)
    iota = ir * 128 + ic

    def bs1(_, c):
        lo, hi = c
        mid = lo + (hi - lo) // 2
        cnt = jnp.sum(jnp.where(sbits >= mid, 1.0, 0.0))
        ge = cnt >= float(_K)
        return (jnp.where(ge, mid, lo), jnp.where(ge, hi, mid))

    lo, _ = jax.lax.fori_loop(0, 31, bs1, (jnp.int32(0), jnp.int32(0x7F800000)))

    cnt_gt = jnp.sum(jnp.where(sbits > lo, 1.0, 0.0))
    need = float(_K) - cnt_gt
    eq = sbits == lo

    def bs2(_, c):
        l2, h2 = c
        mid = l2 + (h2 - l2) // 2
        cc = jnp.sum(jnp.where(eq & (iota <= mid), 1.0, 0.0))
        ge = cc >= need
        return (jnp.where(ge, l2, mid), jnp.where(ge, mid, h2))

    _, tie_hi = jax.lax.fori_loop(
        0, 15, bs2, (jnp.int32(-1), jnp.int32(_NPAD - 1)))

    keep = (sbits > lo) | (eq & (iota <= tie_hi))
    sw_ref[...] = jnp.where(keep, sm, -jnp.inf)


_HCHUNK = _NPAD // 2


def _sc_compact_body(sw_hbm, x1_hbm, y1_hbm, x2_hbm, y2_hbm,
                     outs_hbm, ox1_hbm, oy1_hbm, ox2_hbm, oy2_hbm,
                     sbuf, b0, b1, b2, b3, cs, c0, c1, c2, c3):
    cid = jax.lax.axis_index("c")
    wid = jax.lax.axis_index("s")

    @pl.when((cid == 0) & (wid == 0))
    def _run():
        ninf = jnp.full((16,), _DEAD, jnp.float32)
        for i in range(_CPAD // 16):
            cs[pl.ds(i * 16, 16)] = ninf

        off = jnp.zeros((16,), jnp.int32)
        for half in range(2):
            e0 = half * _HCHUNK
            pltpu.sync_copy(sw_hbm.at[pl.ds(e0, _HCHUNK)], sbuf)
            pltpu.sync_copy(x1_hbm.at[pl.ds(e0, _HCHUNK)], b0)
            pltpu.sync_copy(y1_hbm.at[pl.ds(e0, _HCHUNK)], b1)
            pltpu.sync_copy(x2_hbm.at[pl.ds(e0, _HCHUNK)], b2)
            pltpu.sync_copy(y2_hbm.at[pl.ds(e0, _HCHUNK)], b3)

            def gbody(g, o):
                s = sbuf[pl.ds(g * 16, 16)]
                mask = s > 0.0
                mf = jnp.where(mask, 1.0, 0.0)
                incf = plsc.cumsum(mf)
                pos = o + (incf - mf).astype(jnp.int32)
                plsc.store_scatter(cs, [pos], s, mask=mask)
                plsc.store_scatter(c0, [pos], b0[pl.ds(g * 16, 16)], mask=mask)
                plsc.store_scatter(c1, [pos], b1[pl.ds(g * 16, 16)], mask=mask)
                plsc.store_scatter(c2, [pos], b2[pl.ds(g * 16, 16)], mask=mask)
                plsc.store_scatter(c3, [pos], b3[pl.ds(g * 16, 16)], mask=mask)
                return o + plsc.all_reduce_population_count(mask)

            off = jax.lax.fori_loop(0, _HCHUNK // 16, gbody, off)

        pltpu.sync_copy(cs, outs_hbm)
        pltpu.sync_copy(c0, ox1_hbm)
        pltpu.sync_copy(c1, oy1_hbm)
        pltpu.sync_copy(c2, ox2_hbm)
        pltpu.sync_copy(c3, oy2_hbm)


def _sc_compact(sw2d, x1f, y1f, x2f, y2f):
    mesh = plsc.VectorSubcoreMesh(core_axis_name="c", subcore_axis_name="s")
    f = pl.kernel(
        _sc_compact_body,
        out_type=tuple(
            jax.ShapeDtypeStruct((_CPAD,), jnp.float32) for _ in range(5)),
        mesh=mesh,
        compiler_params=pltpu.CompilerParams(needs_layout_passes=False),
        scratch_types=(
            pltpu.VMEM((_HCHUNK,), jnp.float32),
            pltpu.VMEM((_HCHUNK,), jnp.float32),
            pltpu.VMEM((_HCHUNK,), jnp.float32),
            pltpu.VMEM((_HCHUNK,), jnp.float32),
            pltpu.VMEM((_HCHUNK,), jnp.float32),
            pltpu.VMEM((_CPAD,), jnp.float32),
            pltpu.VMEM((_CPAD,), jnp.float32),
            pltpu.VMEM((_CPAD,), jnp.float32),
            pltpu.VMEM((_CPAD,), jnp.float32),
            pltpu.VMEM((_CPAD,), jnp.float32),
        ),
    )
    return f(sw2d, x1f, y1f, x2f, y2f)


def _combine(av, ai, bv, bi):
    cond = (av > bv) | ((av == bv) & (ai <= bi))
    return jnp.where(cond, av, bv), jnp.where(cond, ai, bi)


def _argmax_splat(v, ix, rolls_ref):
    v01, i01 = _combine(v[0:8], ix[0:8], v[8:16], ix[8:16])
    v23, i23 = _combine(v[16:24], ix[16:24], v[24:32], ix[24:32])
    v45, i45 = _combine(v[32:40], ix[32:40], v[40:48], ix[40:48])
    va, ia = _combine(v01, i01, v23, i23)
    vb, ib = _combine(va, ia, v45, i45)
    for j in range(7):
        r = rolls_ref[j]
        vr = jax.lax.dot(vb, r, precision=jax.lax.Precision.HIGHEST)
        ir = jax.lax.dot(ib, r, precision=jax.lax.Precision.HIGHEST)
        vb, ib = _combine(vb, ib, vr, ir)
    for k in (4, 2, 1):
        vb, ib = _combine(vb, ib, pltpu.roll(vb, k, 0), pltpu.roll(ib, k, 0))
    return vb[0:1], ib[0:1]


def _nms_body(cols_ref, s_ref, rolls_ref, xb1_ref, yb1_ref, xb2_ref, yb2_ref,
              out_ref, sw_ref, a2_ref, iota_ref):
    sw_ref[...] = s_ref[...]

    ir = jax.lax.broadcasted_iota(jnp.int32, (_CROWS, 128), 0)
    ic = jax.lax.broadcasted_iota(jnp.int32, (_CROWS, 128), 1)
    iota_ref[...] = (ir * 128 + ic).astype(jnp.float32)

    x1 = cols_ref[0]
    y1 = cols_ref[1]
    x2 = cols_ref[2]
    y2 = cols_ref[3]
    a2_ref[...] = (x2 - x1) * (y2 - y1)

    li = jax.lax.broadcasted_iota(jnp.int32, (1, 8), 1)

    def step(i, _):
        sw = sw_ref[...]
        m, ixm = _argmax_splat(sw, iota_ref[...], rolls_ref)
        idx = ixm[0, 0].astype(jnp.int32)
        bx1 = xb1_ref[pl.ds(idx, 1), :]
        by1 = yb1_ref[pl.ds(idx, 1), :]
        bx2 = xb2_ref[pl.ds(idx, 1), :]
        by2 = yb2_ref[pl.ds(idx, 1), :]
        alive = m > 0.0
        area1 = (bx2 - bx1) * (by2 - by1)
        ltx = jnp.maximum(bx1, cols_ref[0])
        lty = jnp.maximum(by1, cols_ref[1])
        rbx = jnp.minimum(bx2, cols_ref[2])
        rby = jnp.minimum(by2, cols_ref[3])
        iw = jnp.maximum(rbx - ltx, 0.0)
        ih = jnp.maximum(rby - lty, 0.0)
        inter = iw * ih
        iou = inter / (area1 + a2_ref[...] - inter + 1e-9)
        sw_ref[...] = jnp.where(iou > _IOU_T, _DEAD, sw)
        a8 = alive[:, 0:8]
        row = (jnp.where((li == 0) & a8, bx1[:, 0:8], 0.0)
               + jnp.where((li == 1) & a8, by1[:, 0:8], 0.0)
               + jnp.where((li == 2) & a8, bx2[:, 0:8], 0.0)
               + jnp.where((li == 3) & a8, by2[:, 0:8], 0.0)
               + jnp.where((li == 4) & a8, m[:, 0:8], 0.0))
        out_ref[pl.ds(i, 1), :] = row
        return 0

    jax.lax.fori_loop(0, _OUT, step, 0)


def kernel(boxes, scores):
    boxes_r = jnp.pad(boxes, ((0, _NPAD - _N), (0, 0)))
    colsT = boxes_r.T.reshape(4, _ROWS, 128)
    s2d = jnp.pad(scores, (0, _NPAD - _N)).reshape(_ROWS, 128)

    sw2d = pl.pallas_call(
        _select_body,
        out_shape=jax.ShapeDtypeStruct((_ROWS, 128), jnp.float32),
    )(s2d)

    outs, ox1, oy1, ox2, oy2 = _sc_compact(
        sw2d.reshape(-1), colsT[0].reshape(-1), colsT[1].reshape(-1),
        colsT[2].reshape(-1), colsT[3].reshape(-1))

    csc = outs[:_CN].reshape(_CROWS, 128)
    ccols = jnp.stack([ox1[:_CN], oy1[:_CN], ox2[:_CN], oy2[:_CN]],
                      axis=0).reshape(4, _CROWS, 128)
    lane = jnp.arange(128, dtype=jnp.int32)
    rolls = jnp.stack([
        (lane[:, None] == ((lane[None, :] - (1 << (6 - j))) % 128))
        .astype(jnp.float32)
        for j in range(7)
    ])

    xb1 = jnp.broadcast_to(ox1[:_CN, None], (_CN, 128))
    yb1 = jnp.broadcast_to(oy1[:_CN, None], (_CN, 128))
    xb2 = jnp.broadcast_to(ox2[:_CN, None], (_CN, 128))
    yb2 = jnp.broadcast_to(oy2[:_CN, None], (_CN, 128))

    out = pl.pallas_call(
        _nms_body,
        out_shape=jax.ShapeDtypeStruct((1024, 8), jnp.float32),
        scratch_shapes=[
            pltpu.VMEM((_CROWS, 128), jnp.float32),
            pltpu.VMEM((_CROWS, 128), jnp.float32),
            pltpu.VMEM((_CROWS, 128), jnp.float32),
        ],
    )(ccols, csc, rolls, xb1, yb1, xb2, yb2)
    return out[:_OUT, :5]


# speedup vs baseline: 4.1993x; 4.1993x over previous
import jax
import jax.numpy as jnp
from jax.experimental import pallas as pl
from jax.experimental.pallas import tpu as pltpu
from jax.experimental.pallas import tpu_sc as plsc

_N = 20000
_NPAD = 20480
_ROWS = _NPAD // 128
_K = 6000
_OUT = 1000
_IOU_T = 0.7
_DEAD = -3.0e38

_CN = 6144
_CROWS = _CN // 128
_CPAD = 6400
_TRASHP = 6336
_RFLAT = _CPAD * 4
_TRASHR = 25592
_NWK = 16
_WR = _ROWS // _NWK
_FILL = _CPAD // _NWK


def _select_body(s_ref, sw_ref):
    s = s_ref[...]
    sm = jnp.where(s > 0.0, s, -jnp.inf)
    sbits = jax.lax.bitcast_convert_type(sm, jnp.int32)

    ir = jax.lax.broadcasted_iota(jnp.int32, (_ROWS, 128), 0)
    ic = jax.lax.broadcasted_iota(jnp.int32, (_ROWS, 128), 1)
    iota = ir * 128 + ic

    def bs1(_, c):
        lo, hi = c
        mid = lo + (hi - lo) // 2
        cnt = jnp.sum(jnp.where(sbits >= mid, 1.0, 0.0))
        ge = cnt >= float(_K)
        return (jnp.where(ge, mid, lo), jnp.where(ge, hi, mid))

    lo, _ = jax.lax.fori_loop(0, 31, bs1, (jnp.int32(0), jnp.int32(0x7F800000)))

    cnt_gt = jnp.sum(jnp.where(sbits > lo, 1.0, 0.0))
    need = float(_K) - cnt_gt
    eq = sbits == lo

    def bs2(_, c):
        l2, h2 = c
        mid = l2 + (h2 - l2) // 2
        cc = jnp.sum(jnp.where(eq & (iota <= mid), 1.0, 0.0))
        ge = cc >= need
        return (jnp.where(ge, l2, mid), jnp.where(ge, mid, h2))

    _, tie_hi = jax.lax.fori_loop(
        0, 15, bs2, (jnp.int32(-1), jnp.int32(_NPAD - 1)))

    keep = (sbits > lo) | (eq & (iota <= tie_hi))
    sw_ref[...] = jnp.where(keep, sm, -jnp.inf)


_HCHUNK = _NPAD // 2


def _sc_compact_body(sw_hbm, x1_hbm, y1_hbm, x2_hbm, y2_hbm,
                     outs_hbm, ox1_hbm, oy1_hbm, ox2_hbm, oy2_hbm,
                     sbuf, b0, b1, b2, b3, cs, c0, c1, c2, c3):
    cid = jax.lax.axis_index("c")
    wid = jax.lax.axis_index("s")

    @pl.when((cid == 0) & (wid == 0))
    def _run():
        ninf = jnp.full((16,), _DEAD, jnp.float32)
        for i in range(_CPAD // 16):
            cs[pl.ds(i * 16, 16)] = ninf

        off = jnp.zeros((16,), jnp.int32)
        for half in range(2):
            e0 = half * _HCHUNK
            pltpu.sync_copy(sw_hbm.at[pl.ds(e0, _HCHUNK)], sbuf)
            pltpu.sync_copy(x1_hbm.at[pl.ds(e0, _HCHUNK)], b0)
            pltpu.sync_copy(y1_hbm.at[pl.ds(e0, _HCHUNK)], b1)
            pltpu.sync_copy(x2_hbm.at[pl.ds(e0, _HCHUNK)], b2)
            pltpu.sync_copy(y2_hbm.at[pl.ds(e0, _HCHUNK)], b3)

            def gbody(g, o):
                s = sbuf[pl.ds(g * 16, 16)]
                mask = s > 0.0
                mf = jnp.where(mask, 1.0, 0.0)
                incf = plsc.cumsum(mf)
                pos = o + (incf - mf).astype(jnp.int32)
                plsc.store_scatter(cs, [pos], s, mask=mask)
                plsc.store_scatter(c0, [pos], b0[pl.ds(g * 16, 16)], mask=mask)
                plsc.store_scatter(c1, [pos], b1[pl.ds(g * 16, 16)], mask=mask)
                plsc.store_scatter(c2, [pos], b2[pl.ds(g * 16, 16)], mask=mask)
                plsc.store_scatter(c3, [pos], b3[pl.ds(g * 16, 16)], mask=mask)
                return o + plsc.all_reduce_population_count(mask)

            off = jax.lax.fori_loop(0, _HCHUNK // 16, gbody, off)

        pltpu.sync_copy(cs, outs_hbm)
        pltpu.sync_copy(c0, ox1_hbm)
        pltpu.sync_copy(c1, oy1_hbm)
        pltpu.sync_copy(c2, ox2_hbm)
        pltpu.sync_copy(c3, oy2_hbm)


def _sc_compact(sw2d, x1f, y1f, x2f, y2f):
    mesh = plsc.VectorSubcoreMesh(core_axis_name="c", subcore_axis_name="s")
    f = pl.kernel(
        _sc_compact_body,
        out_type=tuple(
            jax.ShapeDtypeStruct((_CPAD,), jnp.float32) for _ in range(5)),
        mesh=mesh,
        compiler_params=pltpu.CompilerParams(needs_layout_passes=False),
        scratch_types=(
            pltpu.VMEM((_HCHUNK,), jnp.float32),
            pltpu.VMEM((_HCHUNK,), jnp.float32),
            pltpu.VMEM((_HCHUNK,), jnp.float32),
            pltpu.VMEM((_HCHUNK,), jnp.float32),
            pltpu.VMEM((_HCHUNK,), jnp.float32),
            pltpu.VMEM((_CPAD,), jnp.float32),
            pltpu.VMEM((_CPAD,), jnp.float32),
            pltpu.VMEM((_CPAD,), jnp.float32),
            pltpu.VMEM((_CPAD,), jnp.float32),
            pltpu.VMEM((_CPAD,), jnp.float32),
        ),
    )
    return f(sw2d, x1f, y1f, x2f, y2f)


def _combine(av, ai, bv, bi):
    cond = (av > bv) | ((av == bv) & (ai <= bi))
    return jnp.where(cond, av, bv), jnp.where(cond, ai, bi)


def _tree_combine(cands):
    while len(cands) > 1:
        nxt = [(_combine(*cands[i], *cands[i + 1]))
               for i in range(0, len(cands) - 1, 2)]
        if len(cands) % 2:
            nxt.append(cands[-1])
        cands = nxt
    return cands[0]


def _argmax_splat(v, ix):
    v01, i01 = _combine(v[0:8], ix[0:8], v[8:16], ix[8:16])
    v23, i23 = _combine(v[16:24], ix[16:24], v[24:32], ix[24:32])
    v45, i45 = _combine(v[32:40], ix[32:40], v[40:48], ix[40:48])
    va, ia = _combine(v01, i01, v23, i23)
    vb, ib = _combine(va, ia, v45, i45)
    cands = [(vb, ib)] + [(pltpu.roll(vb, k, 0), pltpu.roll(ib, k, 0))
                          for k in range(1, 8)]
    vb, ib = _tree_combine(cands)
    cands = [(vb, ib)] + [(pltpu.roll(vb, k, 1), pltpu.roll(ib, k, 1))
                          for k in range(1, 16)]
    vb, ib = _tree_combine(cands)
    cands = [(vb, ib)] + [(pltpu.roll(vb, k, 1), pltpu.roll(ib, k, 1))
                          for k in range(16, 128, 16)]
    vb, ib = _tree_combine(cands)
    return vb[0:1], ib[0:1]


def _nms_body(cols_ref, s_ref, xb1_ref, yb1_ref, xb2_ref, yb2_ref,
              out_ref, sw_ref, a2_ref, iota_ref):
    sw_ref[...] = s_ref[...]

    ir = jax.lax.broadcasted_iota(jnp.int32, (_CROWS, 128), 0)
    ic = jax.lax.broadcasted_iota(jnp.int32, (_CROWS, 128), 1)
    iota_ref[...] = (ir * 128 + ic).astype(jnp.float32)

    x1 = cols_ref[0]
    y1 = cols_ref[1]
    x2 = cols_ref[2]
    y2 = cols_ref[3]
    a2_ref[...] = (x2 - x1) * (y2 - y1)

    li = jax.lax.broadcasted_iota(jnp.int32, (1, 8), 1)

    def step(i, _):
        sw = sw_ref[...]
        m, ixm = _argmax_splat(sw, iota_ref[...])
        idx = ixm[0, 0].astype(jnp.int32)
        bx1 = xb1_ref[pl.ds(idx, 1), :]
        by1 = yb1_ref[pl.ds(idx, 1), :]
        bx2 = xb2_ref[pl.ds(idx, 1), :]
        by2 = yb2_ref[pl.ds(idx, 1), :]
        alive = m > 0.0
        area1 = (bx2 - bx1) * (by2 - by1)
        ltx = jnp.maximum(bx1, cols_ref[0])
        lty = jnp.maximum(by1, cols_ref[1])
        rbx = jnp.minimum(bx2, cols_ref[2])
        rby = jnp.minimum(by2, cols_ref[3])
        iw = jnp.maximum(rbx - ltx, 0.0)
        ih = jnp.maximum(rby - lty, 0.0)
        inter = iw * ih
        iou = inter / (area1 + a2_ref[...] - inter + 1e-9)
        sw_ref[...] = jnp.where(iou > _IOU_T, _DEAD, sw)
        a8 = alive[:, 0:8]
        row = (jnp.where((li == 0) & a8, bx1[:, 0:8], 0.0)
               + jnp.where((li == 1) & a8, by1[:, 0:8], 0.0)
               + jnp.where((li == 2) & a8, bx2[:, 0:8], 0.0)
               + jnp.where((li == 3) & a8, by2[:, 0:8], 0.0)
               + jnp.where((li == 4) & a8, m[:, 0:8], 0.0))
        out_ref[pl.ds(i, 1), :] = row
        return 0

    jax.lax.fori_loop(0, _OUT, step, 0)


def kernel(boxes, scores):
    boxes_r = jnp.pad(boxes, ((0, _NPAD - _N), (0, 0)))
    colsT = boxes_r.T.reshape(4, _ROWS, 128)
    s2d = jnp.pad(scores, (0, _NPAD - _N)).reshape(_ROWS, 128)

    sw2d = pl.pallas_call(
        _select_body,
        out_shape=jax.ShapeDtypeStruct((_ROWS, 128), jnp.float32),
    )(s2d)

    outs, ox1, oy1, ox2, oy2 = _sc_compact(
        sw2d.reshape(-1), colsT[0].reshape(-1), colsT[1].reshape(-1),
        colsT[2].reshape(-1), colsT[3].reshape(-1))

    csc = outs[:_CN].reshape(_CROWS, 128)
    ccols = jnp.stack([ox1[:_CN], oy1[:_CN], ox2[:_CN], oy2[:_CN]],
                      axis=0).reshape(4, _CROWS, 128)
    xb1 = jnp.broadcast_to(ox1[:_CN, None], (_CN, 128))
    yb1 = jnp.broadcast_to(oy1[:_CN, None], (_CN, 128))
    xb2 = jnp.broadcast_to(ox2[:_CN, None], (_CN, 128))
    yb2 = jnp.broadcast_to(oy2[:_CN, None], (_CN, 128))

    out = pl.pallas_call(
        _nms_body,
        out_shape=jax.ShapeDtypeStruct((1024, 8), jnp.float32),
        scratch_shapes=[
            pltpu.VMEM((_CROWS, 128), jnp.float32),
            pltpu.VMEM((_CROWS, 128), jnp.float32),
            pltpu.VMEM((_CROWS, 128), jnp.float32),
        ],
    )(ccols, csc, xb1, yb1, xb2, yb2)
    return out[:_OUT, :5]


# speedup vs baseline: 4.2573x; 1.0138x over previous
import jax
import jax.numpy as jnp
from jax.experimental import pallas as pl
from jax.experimental.pallas import tpu as pltpu
from jax.experimental.pallas import tpu_sc as plsc

_N = 20000
_NPAD = 20480
_ROWS = _NPAD // 128
_K = 6000
_OUT = 1000
_IOU_T = 0.7
_DEAD = -3.0e38

_CN = 6144
_CROWS = _CN // 128
_CPAD = 6400


def _select_body(s_ref, sw_ref):
    s = s_ref[...]
    sm = jnp.where(s > 0.0, s, -jnp.inf)
    sbits = jax.lax.bitcast_convert_type(sm, jnp.int32)

    ir = jax.lax.broadcasted_iota(jnp.int32, (_ROWS, 128), 0)
    ic = jax.lax.broadcasted_iota(jnp.int32, (_ROWS, 128), 1)
    iota = ir * 128 + ic

    def bs1(_, c):
        lo, hi = c
        mid = lo + (hi - lo) // 2
        cnt = jnp.sum(jnp.where(sbits >= mid, 1.0, 0.0))
        ge = cnt >= float(_K)
        return (jnp.where(ge, mid, lo), jnp.where(ge, hi, mid))

    lo, _ = jax.lax.fori_loop(0, 31, bs1, (jnp.int32(0), jnp.int32(0x7F800000)))

    cnt_gt = jnp.sum(jnp.where(sbits > lo, 1.0, 0.0))
    need = float(_K) - cnt_gt
    eq = sbits == lo

    def bs2(_, c):
        l2, h2 = c
        mid = l2 + (h2 - l2) // 2
        cc = jnp.sum(jnp.where(eq & (iota <= mid), 1.0, 0.0))
        ge = cc >= need
        return (jnp.where(ge, l2, mid), jnp.where(ge, mid, h2))

    _, tie_hi = jax.lax.fori_loop(
        0, 15, bs2, (jnp.int32(-1), jnp.int32(_NPAD - 1)))

    keep = (sbits > lo) | (eq & (iota <= tie_hi))
    sw_ref[...] = jnp.where(keep, sm, -jnp.inf)


_HCHUNK = _NPAD // 2


def _sc_compact_body(sw_hbm, x1_hbm, y1_hbm, x2_hbm, y2_hbm,
                     outs_hbm, ox1_hbm, oy1_hbm, ox2_hbm, oy2_hbm,
                     sbuf, b0, b1, b2, b3, cs, c0, c1, c2, c3):
    cid = jax.lax.axis_index("c")
    wid = jax.lax.axis_index("s")

    @pl.when((cid == 0) & (wid == 0))
    def _run():
        ninf = jnp.full((16,), _DEAD, jnp.float32)
        for i in range(_CPAD // 16):
            cs[pl.ds(i * 16, 16)] = ninf

        off = jnp.zeros((16,), jnp.int32)
        for half in range(2):
            e0 = half * _HCHUNK
            pltpu.sync_copy(sw_hbm.at[pl.ds(e0, _HCHUNK)], sbuf)
            pltpu.sync_copy(x1_hbm.at[pl.ds(e0, _HCHUNK)], b0)
            pltpu.sync_copy(y1_hbm.at[pl.ds(e0, _HCHUNK)], b1)
            pltpu.sync_copy(x2_hbm.at[pl.ds(e0, _HCHUNK)], b2)
            pltpu.sync_copy(y2_hbm.at[pl.ds(e0, _HCHUNK)], b3)

            def gbody(g, o):
                s = sbuf[pl.ds(g * 16, 16)]
                mask = s > 0.0
                mf = jnp.where(mask, 1.0, 0.0)
                incf = plsc.cumsum(mf)
                pos = o + (incf - mf).astype(jnp.int32)
                plsc.store_scatter(cs, [pos], s, mask=mask)
                plsc.store_scatter(c0, [pos], b0[pl.ds(g * 16, 16)], mask=mask)
                plsc.store_scatter(c1, [pos], b1[pl.ds(g * 16, 16)], mask=mask)
                plsc.store_scatter(c2, [pos], b2[pl.ds(g * 16, 16)], mask=mask)
                plsc.store_scatter(c3, [pos], b3[pl.ds(g * 16, 16)], mask=mask)
                return o + plsc.all_reduce_population_count(mask)

            off = jax.lax.fori_loop(0, _HCHUNK // 16, gbody, off)

        pltpu.sync_copy(cs, outs_hbm)
        pltpu.sync_copy(c0, ox1_hbm)
        pltpu.sync_copy(c1, oy1_hbm)
        pltpu.sync_copy(c2, ox2_hbm)
        pltpu.sync_copy(c3, oy2_hbm)


def _sc_compact(sw2d, x1f, y1f, x2f, y2f):
    mesh = plsc.VectorSubcoreMesh(core_axis_name="c", subcore_axis_name="s")
    f = pl.kernel(
        _sc_compact_body,
        out_type=tuple(
            jax.ShapeDtypeStruct((_CPAD,), jnp.float32) for _ in range(5)),
        mesh=mesh,
        compiler_params=pltpu.CompilerParams(needs_layout_passes=False),
        scratch_types=(
            pltpu.VMEM((_HCHUNK,), jnp.float32),
            pltpu.VMEM((_HCHUNK,), jnp.float32),
            pltpu.VMEM((_HCHUNK,), jnp.float32),
            pltpu.VMEM((_HCHUNK,), jnp.float32),
            pltpu.VMEM((_HCHUNK,), jnp.float32),
            pltpu.VMEM((_CPAD,), jnp.float32),
            pltpu.VMEM((_CPAD,), jnp.float32),
            pltpu.VMEM((_CPAD,), jnp.float32),
            pltpu.VMEM((_CPAD,), jnp.float32),
            pltpu.VMEM((_CPAD,), jnp.float32),
        ),
    )
    return f(sw2d, x1f, y1f, x2f, y2f)


def _combine(av, ai, bv, bi):
    cond = (av > bv) | ((av == bv) & (ai <= bi))
    return jnp.where(cond, av, bv), jnp.where(cond, ai, bi)


def _tree_combine(cands):
    while len(cands) > 1:
        nxt = [(_combine(*cands[i], *cands[i + 1]))
               for i in range(0, len(cands) - 1, 2)]
        if len(cands) % 2:
            nxt.append(cands[-1])
        cands = nxt
    return cands[0]


def _argmax_splat(v, ix):
    v01, i01 = _combine(v[0:8], ix[0:8], v[8:16], ix[8:16])
    v23, i23 = _combine(v[16:24], ix[16:24], v[24:32], ix[24:32])
    v45, i45 = _combine(v[32:40], ix[32:40], v[40:48], ix[40:48])
    va, ia = _combine(v01, i01, v23, i23)
    vb, ib = _combine(va, ia, v45, i45)
    cands = [(vb, ib)] + [(pltpu.roll(vb, k, 0), pltpu.roll(ib, k, 0))
                          for k in range(1, 8)]
    vb, ib = _tree_combine(cands)
    cands = [(vb, ib)] + [(pltpu.roll(vb, k, 1), pltpu.roll(ib, k, 1))
                          for k in range(1, 16)]
    vb, ib = _tree_combine(cands)
    cands = [(vb, ib)] + [(pltpu.roll(vb, k, 1), pltpu.roll(ib, k, 1))
                          for k in range(16, 128, 16)]
    vb, ib = _tree_combine(cands)
    return vb[0:1], ib[0:1]


def _nms_body(cols_ref, s_ref, xb1_ref, yb1_ref, xb2_ref, yb2_ref,
              out_ref, sw_ref, a2_ref, iota_ref):
    sw_ref[...] = s_ref[...]

    ir = jax.lax.broadcasted_iota(jnp.int32, (_CROWS, 128), 0)
    ic = jax.lax.broadcasted_iota(jnp.int32, (_CROWS, 128), 1)
    iota_ref[...] = (ir * 128 + ic).astype(jnp.float32)

    x1 = cols_ref[0]
    y1 = cols_ref[1]
    x2 = cols_ref[2]
    y2 = cols_ref[3]
    a2_ref[...] = (x2 - x1) * (y2 - y1)

    li = jax.lax.broadcasted_iota(jnp.int32, (1, 8), 1)

    def step(i, _):
        sw = sw_ref[...]
        m, ixm = _argmax_splat(sw, iota_ref[...])
        idx = ixm[0, 0].astype(jnp.int32)
        bx1 = xb1_ref[pl.ds(idx, 1), :]
        by1 = yb1_ref[pl.ds(idx, 1), :]
        bx2 = xb2_ref[pl.ds(idx, 1), :]
        by2 = yb2_ref[pl.ds(idx, 1), :]
        alive = m > 0.0
        area1 = (bx2 - bx1) * (by2 - by1)
        ltx = jnp.maximum(bx1, cols_ref[0])
        lty = jnp.maximum(by1, cols_ref[1])
        rbx = jnp.minimum(bx2, cols_ref[2])
        rby = jnp.minimum(by2, cols_ref[3])
        iw = jnp.maximum(rbx - ltx, 0.0)
        ih = jnp.maximum(rby - lty, 0.0)
        inter = iw * ih
        iou = inter / (area1 + a2_ref[...] - inter + 1e-9)
        sw_ref[...] = jnp.where(iou > _IOU_T, _DEAD, sw)
        a8 = alive[:, 0:8]
        row = (jnp.where((li == 0) & a8, bx1[:, 0:8], 0.0)
               + jnp.where((li == 1) & a8, by1[:, 0:8], 0.0)
               + jnp.where((li == 2) & a8, bx2[:, 0:8], 0.0)
               + jnp.where((li == 3) & a8, by2[:, 0:8], 0.0)
               + jnp.where((li == 4) & a8, m[:, 0:8], 0.0))
        out_ref[pl.ds(i, 1), :] = row
        return 0

    jax.lax.fori_loop(0, _OUT, step, 0)


def kernel(boxes, scores):
    boxes_r = jnp.pad(boxes, ((0, _NPAD - _N), (0, 0)))
    colsT = boxes_r.T.reshape(4, _ROWS, 128)
    s2d = jnp.pad(scores, (0, _NPAD - _N)).reshape(_ROWS, 128)

    sw2d = pl.pallas_call(
        _select_body,
        out_shape=jax.ShapeDtypeStruct((_ROWS, 128), jnp.float32),
    )(s2d)

    outs, ox1, oy1, ox2, oy2 = _sc_compact(
        sw2d.reshape(-1), colsT[0].reshape(-1), colsT[1].reshape(-1),
        colsT[2].reshape(-1), colsT[3].reshape(-1))

    csc = outs[:_CN].reshape(_CROWS, 128)
    ccols = jnp.stack([ox1[:_CN], oy1[:_CN], ox2[:_CN], oy2[:_CN]],
                      axis=0).reshape(4, _CROWS, 128)
    xb1 = jnp.broadcast_to(ox1[:_CN, None], (_CN, 128))
    yb1 = jnp.broadcast_to(oy1[:_CN, None], (_CN, 128))
    xb2 = jnp.broadcast_to(ox2[:_CN, None], (_CN, 128))
    yb2 = jnp.broadcast_to(oy2[:_CN, None], (_CN, 128))

    out = pl.pallas_call(
        _nms_body,
        out_shape=jax.ShapeDtypeStruct((1024, 8), jnp.float32),
        scratch_shapes=[
            pltpu.VMEM((_CROWS, 128), jnp.float32),
            pltpu.VMEM((_CROWS, 128), jnp.float32),
            pltpu.VMEM((_CROWS, 128), jnp.float32),
        ],
    )(ccols, csc, xb1, yb1, xb2, yb2)
    return out[:_OUT, :5]
